# baseline (speedup 1.0000x reference)
import jax
import jax.numpy as jnp
from jax.experimental import pallas as pl

_BR = 256
_BC = 2048


def _body(w_ref, t_ref, o_ref):
    s = jax.nn.sigmoid(t_ref[:])
    w = w_ref[:]
    o_ref[:] = jnp.sign(w) * jnp.maximum(jnp.abs(w) - s, 0.0)


def kernel(weight, threshold):
    n_rows, n_cols = weight.shape
    return pl.pallas_call(
        _body,
        grid=(n_rows // _BR, n_cols // _BC),
        in_specs=[
            pl.BlockSpec((_BR, _BC), lambda i, j: (i, j)),
            pl.BlockSpec((_BR, 1), lambda i, j: (i, 0)),
        ],
        out_specs=pl.BlockSpec((_BR, _BC), lambda i, j: (i, j)),
        out_shape=jax.ShapeDtypeStruct((n_rows, n_cols), weight.dtype),
    )(weight, threshold)


# speedup vs baseline: 1.2796x; 1.2796x over previous
import jax
import jax.numpy as jnp
from jax.experimental import pallas as pl

_BR = 256
_BC = 2048


def _body(w_ref, t_ref, o_ref):
    s = jax.nn.sigmoid(t_ref[:])
    w = w_ref[:]
    o_ref[:] = w - jnp.minimum(jnp.maximum(w, -s), s)


def kernel(weight, threshold):
    n_rows, n_cols = weight.shape
    return pl.pallas_call(
        _body,
        grid=(n_rows // _BR, n_cols // _BC),
        in_specs=[
            pl.BlockSpec((_BR, _BC), lambda i, j: (i, j)),
            pl.BlockSpec((_BR, 1), lambda i, j: (i, 0)),
        ],
        out_specs=pl.BlockSpec((_BR, _BC), lambda i, j: (i, j)),
        out_shape=jax.ShapeDtypeStruct((n_rows, n_cols), weight.dtype),
    )(weight, threshold)


# speedup vs baseline: 1.4100x; 1.1019x over previous
import jax
import jax.numpy as jnp
from jax.experimental import pallas as pl

_BR = 128
_BC = 8192


def _body(w_ref, t_ref, o_ref):
    s = jax.nn.sigmoid(t_ref[:])
    w = w_ref[:]
    o_ref[:] = w - jnp.minimum(jnp.maximum(w, -s), s)


def kernel(weight, threshold):
    n_rows, n_cols = weight.shape
    return pl.pallas_call(
        _body,
        grid=(n_rows // _BR, n_cols // _BC),
        in_specs=[
            pl.BlockSpec((_BR, _BC), lambda i, j: (i, j)),
            pl.BlockSpec((_BR, 1), lambda i, j: (i, 0)),
        ],
        out_specs=pl.BlockSpec((_BR, _BC), lambda i, j: (i, j)),
        out_shape=jax.ShapeDtypeStruct((n_rows, n_cols), weight.dtype),
    )(weight, threshold)
